# baseline (speedup 1.0000x reference)
import functools
import jax
import jax.numpy as jnp
from jax import lax
from jax.experimental import pallas as pl
from jax.experimental.pallas import tpu as pltpu
from jax.experimental.pallas import tpu_sc as plsc

D = 64
NW = 32
G = 128
NSUB = G // 128
NBUF = 2


def _emb_kernel(B):
    R = B // NW
    N = R // G
    assert N % NBUF == 0

    mesh = plsc.VectorSubcoreMesh(core_axis_name="c", subcore_axis_name="s")

    @functools.partial(
        pl.kernel,
        mesh=mesh,
        out_type=jax.ShapeDtypeStruct((B, D), jnp.float32),
        scratch_types=[
            pltpu.VMEM((NBUF, NSUB, 128), jnp.int32),
            pltpu.VMEM((NBUF, NSUB, 128), jnp.int32),
            pltpu.VMEM((NBUF, G, 128), jnp.float32),
            pltpu.VMEM((NBUF, G, D), jnp.float32),
            pltpu.SemaphoreType.DMA((NBUF,)),
            pltpu.SemaphoreType.DMA((NBUF,)),
            pltpu.SemaphoreType.DMA((NBUF,)),
        ],
    )
    def k(x_hbm, lut_hbm, out_hbm, idx_v, widx_v, rows_v, ob_v,
          sem_i, sem_g, sem_o):
        wid = lax.axis_index("s") * 2 + lax.axis_index("c")
        cbase = wid * N * NSUB

        def idx_copy(g, b):
            return pltpu.make_async_copy(
                x_hbm.at[pl.ds(cbase + g * NSUB, NSUB)], idx_v.at[b], sem_i.at[b]
            )

        def widx_prep(b):
            for j in range(NSUB):
                for q in range(128 // 16):
                    sl = pl.ds(q * 16, 16)
                    widx_v[b, j, sl] = lax.shift_right_logical(
                        idx_v[b, j, sl], 1
                    )

        def gathers(b):
            return [
                pltpu.make_async_copy(
                    lut_hbm.at[widx_v.at[b, j]],
                    rows_v.at[b, pl.ds(j * 128, 128)],
                    sem_g.at[b],
                )
                for j in range(NSUB)
            ]

        def out_copy(g, b):
            return pltpu.make_async_copy(
                ob_v.at[b], out_hbm.at[pl.ds((cbase + g * NSUB) * 128, G)],
                sem_o.at[b],
            )

        def compute(b):
            def grp_body(q, c2):
                r0 = q * 16
                jq = r0 // 128
                kq = r0 - jq * 128
                xv = idx_v[b, jq, pl.ds(kq, 16)]
                scv = jnp.where(xv == 0, jnp.float32(0.0), jnp.float32(8.0))
                hv = (xv & 1) * 64
                for i in range(16):
                    sc = scv[i]
                    hof = hv[i]
                    r = r0 + i
                    for j in range(D // 16):
                        src = pl.ds(hof + j * 16, 16)
                        dst = pl.ds(j * 16, 16)
                        ob_v[b, r, dst] = rows_v[b, r, src] * sc
                return c2

            lax.fori_loop(0, G // 16, grp_body, 0)

        for b in range(NBUF):
            idx_copy(b, b).start()
        idx_copy(0, 0).wait()
        widx_prep(0)
        for c in gathers(0):
            c.start()

        def outer(o, carry):
            for b in range(NBUF):
                g = o * NBUF + b
                nb = (b + 1) % NBUF
                for c in gathers(b):
                    c.wait()
                @pl.when(g + 1 < N)
                def _():
                    idx_copy(g + 1, nb).wait()
                    widx_prep(nb)

                    @pl.when(g + 1 >= NBUF)
                    def _():
                        out_copy(g + 1 - NBUF, nb).wait()

                    for c in gathers(nb):
                        c.start()

                compute(b)
                out_copy(g, b).start()

                @pl.when(g + NBUF < N)
                def _():
                    idx_copy(g + NBUF, b).start()

            return carry

        lax.fori_loop(0, N // NBUF, outer, 0)

        for b in range(NBUF):
            g = N - NBUF + b
            out_copy(g, b).wait()

    return k


def kernel(x, lut):
    B = x.shape[0] * x.shape[1]
    xr = x.reshape(B // 128, 128)
    lutw = lut.reshape(lut.shape[0] // 2, 2 * D)
    out = _emb_kernel(B)(xr, lutw)
    return out.reshape(x.shape[0], x.shape[1], D)


# speedup vs baseline: 1.0069x; 1.0069x over previous
import functools
import jax
import jax.numpy as jnp
from jax import lax
from jax.experimental import pallas as pl
from jax.experimental.pallas import tpu as pltpu
from jax.experimental.pallas import tpu_sc as plsc

D = 64
NW = 32
G = 256
NSUB = G // 128
NBUF = 2


def _emb_kernel(B):
    R = B // NW
    N = R // G
    assert N % NBUF == 0

    mesh = plsc.VectorSubcoreMesh(core_axis_name="c", subcore_axis_name="s")

    @functools.partial(
        pl.kernel,
        mesh=mesh,
        out_type=jax.ShapeDtypeStruct((B, D), jnp.float32),
        scratch_types=[
            pltpu.VMEM((NBUF, NSUB, 128), jnp.int32),
            pltpu.VMEM((NBUF, NSUB, 128), jnp.int32),
            pltpu.VMEM((NBUF, G, 128), jnp.float32),
            pltpu.VMEM((G, D), jnp.float32),
            pltpu.SemaphoreType.DMA((NBUF,)),
            pltpu.SemaphoreType.DMA((NBUF,)),
            pltpu.SemaphoreType.DMA,
        ],
    )
    def k(x_hbm, lut_hbm, out_hbm, idx_v, widx_v, rows_v, ob_v,
          sem_i, sem_g, sem_o):
        wid = lax.axis_index("s") * 2 + lax.axis_index("c")
        cbase = wid * N * NSUB

        def idx_copy(g, b):
            return pltpu.make_async_copy(
                x_hbm.at[pl.ds(cbase + g * NSUB, NSUB)], idx_v.at[b], sem_i.at[b]
            )

        def widx_prep(b):
            for j in range(NSUB):
                for q in range(128 // 16):
                    sl = pl.ds(q * 16, 16)
                    widx_v[b, j, sl] = lax.shift_right_logical(
                        idx_v[b, j, sl], 1
                    )

        def gathers(b):
            return [
                pltpu.make_async_copy(
                    lut_hbm.at[widx_v.at[b, j]],
                    rows_v.at[b, pl.ds(j * 128, 128)],
                    sem_g.at[b],
                )
                for j in range(NSUB)
            ]

        def out_copy(g):
            return pltpu.make_async_copy(
                ob_v, out_hbm.at[pl.ds((cbase + g * NSUB) * 128, G)],
                sem_o,
            )

        def compute(b):
            def grp_body(q, c2):
                r0 = q * 16
                jq = r0 // 128
                kq = r0 - jq * 128
                xv = idx_v[b, jq, pl.ds(kq, 16)]
                scv = jnp.where(xv == 0, jnp.float32(0.0), jnp.float32(8.0))
                hv = (xv & 1) * 64
                for i in range(16):
                    sc = scv[i]
                    hof = hv[i]
                    r = r0 + i
                    for j in range(D // 16):
                        src = pl.ds(hof + j * 16, 16)
                        dst = pl.ds(j * 16, 16)
                        ob_v[r, dst] = rows_v[b, r, src] * sc
                return c2

            lax.fori_loop(0, G // 16, grp_body, 0)

        for b in range(NBUF):
            idx_copy(b, b).start()
        idx_copy(0, 0).wait()
        widx_prep(0)
        for c in gathers(0):
            c.start()

        def outer(o, carry):
            for b in range(NBUF):
                g = o * NBUF + b
                nb = (b + 1) % NBUF
                for c in gathers(b):
                    c.wait()
                @pl.when(g + 1 < N)
                def _():
                    idx_copy(g + 1, nb).wait()
                    widx_prep(nb)
                    for c in gathers(nb):
                        c.start()

                @pl.when(g >= 1)
                def _():
                    out_copy(g - 1).wait()
                compute(b)
                out_copy(g).start()

                @pl.when(g + NBUF < N)
                def _():
                    idx_copy(g + NBUF, b).start()

            return carry

        lax.fori_loop(0, N // NBUF, outer, 0)

        out_copy(N - 1).wait()

    return k


def kernel(x, lut):
    B = x.shape[0] * x.shape[1]
    xr = x.reshape(B // 128, 128)
    lutw = lut.reshape(lut.shape[0] // 2, 2 * D)
    out = _emb_kernel(B)(xr, lutw)
    return out.reshape(x.shape[0], x.shape[1], D)
